# baseline (speedup 1.0000x reference)
import functools

import jax
import jax.numpy as jnp
from jax import lax
from jax.experimental import pallas as pl
from jax.experimental.pallas import tpu as pltpu
from jax.experimental.pallas import tpu_sc as plsc

N = 10000
NP = 10240
C = 256
H = 128
E = 160000
CK = 128
NT = 16
CHUNKS = 80
EP = NT * CHUNKS * CK
ACCR = 10112
TROWS = 632
DUMMY = 10008
NCLASS = 64
GROUPS = 32
STEPS = 10
BN = 512
NBLK = NP // BN
_HI = lax.Precision.HIGHEST
_f32 = jnp.float32


def _row_spec(r, c):
    return pl.BlockSpec((r, c), lambda i: (i, 0))


def _full_spec(r, c):
    return pl.BlockSpec((r, c), lambda i: (0, 0))


def _tce0_body(x_ref, w_ref, b_ref, u0_ref, u1_ref):
    u = jnp.dot(x_ref[...], w_ref[...], precision=_HI,
                preferred_element_type=_f32) + b_ref[...]
    u0_ref[...] = u[:, :H]
    u1_ref[...] = u[:, H:]


def _tce0(xp, W1, b1row):
    return pl.pallas_call(
        _tce0_body,
        grid=(NBLK,),
        in_specs=[_row_spec(BN, C), _full_spec(C, C), _full_spec(1, C)],
        out_specs=[_row_spec(BN, H), _row_spec(BN, H)],
        out_shape=[jax.ShapeDtypeStruct((NP, H), _f32)] * 2,
    )(xp, W1, b1row)


def _rk4_combine(wk, cz, cs, a0_ref, a1_ref, h_ref, ks_ref, mt_ref):
    agg = jnp.concatenate([a0_ref[...], a1_ref[...]], axis=1)
    k = jnp.maximum(mt_ref[...] * agg, 0.0)
    ksn = ks_ref[...] + wk * k
    z = h_ref[...] + cz * k + cs * ksn
    return z, ksn


def _tcf_body(wk, cz, cs, a0_ref, a1_ref, h_ref, ks_ref, mt_ref,
              g_ref, bt_ref, p_ref, w_ref, bias_ref,
              u0_ref, u1_ref, z_ref, kso_ref):
    z, ksn = _rk4_combine(wk, cz, cs, a0_ref, a1_ref, h_ref, ks_ref, mt_ref)
    kso_ref[...] = ksn
    z_ref[...] = z
    p = p_ref[...]
    m = jnp.dot(z, p, preferred_element_type=_f32)
    q = jnp.dot(z * z, p, preferred_element_type=_f32)
    g = (z - m) * lax.rsqrt(q - m * m + 1e-5) * g_ref[...] + bt_ref[...]
    u = jnp.dot(g, w_ref[...], preferred_element_type=_f32) + bias_ref[...]
    u0_ref[...] = u[:, :H]
    u1_ref[...] = u[:, H:]


def _tcf(wk, cz, cs, a0, a1, h, ks, Mp, gamma_row, beta_row, P, W2h,
         bias_row):
    return pl.pallas_call(
        functools.partial(_tcf_body, wk, cz, cs),
        grid=(NBLK,),
        in_specs=[_row_spec(BN, H), _row_spec(BN, H), _row_spec(BN, C),
                  _row_spec(BN, C), _row_spec(BN, 1), _full_spec(1, C),
                  _full_spec(1, C), _full_spec(C, C), _full_spec(C, C),
                  _full_spec(1, C)],
        out_specs=[_row_spec(BN, H), _row_spec(BN, H), _row_spec(BN, C),
                   _row_spec(BN, C)],
        out_shape=[jax.ShapeDtypeStruct((NP, H), _f32)] * 2
        + [jax.ShapeDtypeStruct((NP, C), _f32)] * 2,
    )(a0, a1, h, ks, Mp, gamma_row, beta_row, P, W2h, bias_row)


def _tcsf_body(a0_ref, a1_ref, h_ref, ks_ref, mt_ref, o_ref):
    dt = 1.0 / STEPS
    z, _ = _rk4_combine(1.0, 0.0, dt / 6, a0_ref, a1_ref, h_ref, ks_ref,
                        mt_ref)
    lg = z[:, :NCLASS]
    m = jnp.max(lg, axis=1, keepdims=True)
    e = jnp.exp(lg - m)
    lse = jnp.log(jnp.sum(e, axis=1, keepdims=True))
    o_ref[...] = lg - m - lse


def _tcsf(a0, a1, h, ks, Mp):
    return pl.pallas_call(
        _tcsf_body,
        grid=(NBLK,),
        in_specs=[_row_spec(BN, H), _row_spec(BN, H), _row_spec(BN, C),
                  _row_spec(BN, C), _row_spec(BN, 1)],
        out_specs=_row_spec(BN, NCLASS),
        out_shape=jax.ShapeDtypeStruct((NP, NCLASS), _f32),
    )(a0, a1, h, ks, Mp)


@functools.partial(
    pl.kernel,
    mesh=plsc.VectorSubcoreMesh(core_axis_name="c", subcore_axis_name="s"),
    out_type=[jax.ShapeDtypeStruct((NP, H), _f32)] * 2,
    scratch_types=[
        pltpu.VMEM((CHUNKS, CK), jnp.int32),
        pltpu.VMEM((CHUNKS, CK), jnp.int32),
        pltpu.VMEM((CK, H), _f32),
        pltpu.VMEM_SHARED((ACCR, H), _f32),
        pltpu.SemaphoreType.DMA,
    ],
)
def _sc_seg(u0, u1, srcg, tgtg, zrows, agg0, agg1,
            src_buf, tgt_buf, rows, acc, gsem):
    c = lax.axis_index("c")
    s = lax.axis_index("s")

    pltpu.sync_copy(srcg.at[s], src_buf)
    pltpu.sync_copy(tgtg.at[s], tgt_buf)

    r0 = s * TROWS
    pltpu.sync_copy(zrows, acc.at[pl.ds(r0, TROWS)])
    plsc.subcore_barrier()

    def _run(table):
        def body(ci, carry):
            pltpu.async_copy(table.at[src_buf.at[ci]], rows, gsem).wait()
            pltpu.sync_copy(rows, acc.at[tgt_buf.at[ci]], add=True)
            return carry
        lax.fori_loop(0, CHUNKS, body, 0)

    @pl.when(c == 0)
    def _():
        _run(u0)

    @pl.when(c == 1)
    def _():
        _run(u1)

    plsc.subcore_barrier()

    def _wb(agg):
        pltpu.sync_copy(acc.at[pl.ds(r0, TROWS)], agg.at[pl.ds(r0, TROWS)])

    @pl.when(c == 0)
    def _():
        _wb(agg0)

    @pl.when(c == 1)
    def _():
        _wb(agg1)


def kernel(x, src, tgt, Mtgt, W1, b1, gamma, beta, W2, b2):
    xp = jnp.zeros((NP, C), _f32).at[:N].set(x)
    Mp = jnp.zeros((NP, 1), _f32).at[:N].set(Mtgt)
    src = src.astype(jnp.int32)
    perm = jnp.argsort(src)
    src = src[perm]
    tgt = tgt[perm]
    srcg = (jnp.zeros((EP,), jnp.int32).at[:E].set(src.astype(jnp.int32))
            .reshape(NT, CHUNKS, CK))
    tgtg = (jnp.full((EP,), DUMMY, jnp.int32).at[:E].set(tgt.astype(jnp.int32))
            .reshape(NT, CHUNKS, CK))

    gsz = C // GROUPS
    G = jnp.kron(jnp.eye(GROUPS, dtype=_f32), jnp.ones((gsz, 1), _f32))
    P = (G @ G.T) / gsz
    gamma_row = gamma.reshape(1, C)
    beta_row = beta.reshape(1, C)
    b1row = b1.reshape(1, C)
    W2h = W2[1:]
    w20 = W2[0]
    dt = 1.0 / STEPS
    zeros_nc = jnp.zeros((NP, C), _f32)
    zrows = jnp.zeros((TROWS, H), _f32)

    def bias(tt):
        return (tt * w20 + b2).reshape(1, C)

    u0, u1 = _tce0(xp, W1, b1row)
    a0, a1 = _sc_seg(u0, u1, srcg, tgtg, zrows)
    u0, u1, h, ks = _tcf(1.0, 1.0, 0.0, a0, a1, zeros_nc, zeros_nc, Mp,
                         gamma_row, beta_row, P, W2h, bias(0.0))

    for i in range(STEPS):
        t = i * dt
        ks = zeros_nc
        for (wk, cz, cs), tb in [
            ((1.0, dt / 2, 0.0), t + dt / 2),
            ((2.0, dt / 2, 0.0), t + dt / 2),
            ((2.0, dt, 0.0), t + dt),
            ((1.0, 0.0, dt / 6), t + dt),
        ]:
            a0, a1 = _sc_seg(u0, u1, srcg, tgtg, zrows)
            if i == STEPS - 1 and cs != 0.0:
                return _tcsf(a0, a1, h, ks, Mp)[:N]
            hin = h
            u0, u1, z, ks = _tcf(wk, cz, cs, a0, a1, hin, ks, Mp,
                                 gamma_row, beta_row, P, W2h, bias(tb))
            if cs != 0.0:
                h = z


# speedup vs baseline: 1.1431x; 1.1431x over previous
import functools

import jax
import jax.numpy as jnp
from jax import lax
from jax.experimental import pallas as pl
from jax.experimental.pallas import tpu as pltpu
from jax.experimental.pallas import tpu_sc as plsc

N = 10000
NP = 10240
C = 256
H = 128
E = 160000
CK = 128
NT = 16
CHUNKS = 80
EP = NT * CHUNKS * CK
ACCR = 10112
TROWS = 632
DUMMY = 10008
NCLASS = 64
GROUPS = 32
STEPS = 10
BN = 512
NBLK = NP // BN
_HI = lax.Precision.HIGHEST
_f32 = jnp.float32


def _row_spec(r, c):
    return pl.BlockSpec((r, c), lambda i: (i, 0))


def _full_spec(r, c):
    return pl.BlockSpec((r, c), lambda i: (0, 0))


def _tce0_body(x_ref, w_ref, b_ref, u0_ref, u1_ref):
    u = jnp.dot(x_ref[...], w_ref[...], precision=_HI,
                preferred_element_type=_f32) + b_ref[...]
    u0_ref[...] = u[:, :H]
    u1_ref[...] = u[:, H:]


def _tce0(xp, W1, b1row):
    return pl.pallas_call(
        _tce0_body,
        grid=(NBLK,),
        in_specs=[_row_spec(BN, C), _full_spec(C, C), _full_spec(1, C)],
        out_specs=[_row_spec(BN, H), _row_spec(BN, H)],
        out_shape=[jax.ShapeDtypeStruct((NP, H), _f32)] * 2,
    )(xp, W1, b1row)


def _rk4_combine(wk, cz, cs, a0_ref, a1_ref, h_ref, ks_ref, mt_ref):
    agg = jnp.concatenate([a0_ref[...], a1_ref[...]], axis=1)
    k = jnp.maximum(mt_ref[...] * agg, 0.0)
    ksn = ks_ref[...] + wk * k
    z = h_ref[...] + cz * k + cs * ksn
    return z, ksn


def _tcf_body(wk, cz, cs, a0_ref, a1_ref, h_ref, ks_ref, mt_ref,
              g_ref, bt_ref, p_ref, w_ref, bias_ref,
              u0_ref, u1_ref, z_ref, kso_ref):
    z, ksn = _rk4_combine(wk, cz, cs, a0_ref, a1_ref, h_ref, ks_ref, mt_ref)
    kso_ref[...] = ksn
    z_ref[...] = z
    p = p_ref[...]
    m = jnp.dot(z, p, preferred_element_type=_f32)
    q = jnp.dot(z * z, p, preferred_element_type=_f32)
    g = (z - m) * lax.rsqrt(q - m * m + 1e-5) * g_ref[...] + bt_ref[...]
    u = jnp.dot(g, w_ref[...], preferred_element_type=_f32) + bias_ref[...]
    u0_ref[...] = u[:, :H]
    u1_ref[...] = u[:, H:]


def _tcf(wk, cz, cs, a0, a1, h, ks, Mp, gamma_row, beta_row, P, W2h,
         bias_row):
    return pl.pallas_call(
        functools.partial(_tcf_body, wk, cz, cs),
        grid=(NBLK,),
        in_specs=[_row_spec(BN, H), _row_spec(BN, H), _row_spec(BN, C),
                  _row_spec(BN, C), _row_spec(BN, 1), _full_spec(1, C),
                  _full_spec(1, C), _full_spec(C, C), _full_spec(C, C),
                  _full_spec(1, C)],
        out_specs=[_row_spec(BN, H), _row_spec(BN, H), _row_spec(BN, C),
                   _row_spec(BN, C)],
        out_shape=[jax.ShapeDtypeStruct((NP, H), _f32)] * 2
        + [jax.ShapeDtypeStruct((NP, C), _f32)] * 2,
    )(a0, a1, h, ks, Mp, gamma_row, beta_row, P, W2h, bias_row)


def _tcsf_body(a0_ref, a1_ref, h_ref, ks_ref, mt_ref, o_ref):
    dt = 1.0 / STEPS
    z, _ = _rk4_combine(1.0, 0.0, dt / 6, a0_ref, a1_ref, h_ref, ks_ref,
                        mt_ref)
    lg = z[:, :NCLASS]
    m = jnp.max(lg, axis=1, keepdims=True)
    e = jnp.exp(lg - m)
    lse = jnp.log(jnp.sum(e, axis=1, keepdims=True))
    o_ref[...] = lg - m - lse


def _tcsf(a0, a1, h, ks, Mp):
    return pl.pallas_call(
        _tcsf_body,
        grid=(NBLK,),
        in_specs=[_row_spec(BN, H), _row_spec(BN, H), _row_spec(BN, C),
                  _row_spec(BN, C), _row_spec(BN, 1)],
        out_specs=_row_spec(BN, NCLASS),
        out_shape=jax.ShapeDtypeStruct((NP, NCLASS), _f32),
    )(a0, a1, h, ks, Mp)


@functools.partial(
    pl.kernel,
    mesh=plsc.VectorSubcoreMesh(core_axis_name="c", subcore_axis_name="s"),
    out_type=[jax.ShapeDtypeStruct((NP, H), _f32)] * 2,
    scratch_types=[
        pltpu.VMEM((CHUNKS, CK), jnp.int32),
        pltpu.VMEM((CHUNKS, CK), jnp.int32),
        pltpu.VMEM((CK, H), _f32),
        pltpu.VMEM_SHARED((ACCR, H), _f32),
        pltpu.SemaphoreType.DMA,
    ],
)
def _sc_seg(u0, u1, srcg, tgtg, zrows, agg0, agg1,
            src_buf, tgt_buf, rows, acc, gsem):
    c = lax.axis_index("c")
    s = lax.axis_index("s")

    pltpu.sync_copy(srcg.at[s], src_buf)
    pltpu.sync_copy(tgtg.at[s], tgt_buf)

    r0 = s * TROWS
    pltpu.sync_copy(zrows, acc.at[pl.ds(r0, TROWS)])
    plsc.subcore_barrier()

    def _run(table):
        def body(ci, carry):
            pltpu.async_copy(table.at[src_buf.at[ci]], rows, gsem).wait()
            pltpu.sync_copy(rows, acc.at[tgt_buf.at[ci]], add=True)
            return carry
        lax.fori_loop(0, CHUNKS, body, 0)

    @pl.when(c == 0)
    def _():
        _run(u0)

    @pl.when(c == 1)
    def _():
        _run(u1)

    plsc.subcore_barrier()

    def _wb(agg):
        pltpu.sync_copy(acc.at[pl.ds(r0, TROWS)], agg.at[pl.ds(r0, TROWS)])

    @pl.when(c == 0)
    def _():
        _wb(agg0)

    @pl.when(c == 1)
    def _():
        _wb(agg1)


def kernel(x, src, tgt, Mtgt, W1, b1, gamma, beta, W2, b2):
    xp = jnp.zeros((NP, C), _f32).at[:N].set(x)
    Mp = jnp.zeros((NP, 1), _f32).at[:N].set(Mtgt)
    srcg = (jnp.zeros((EP,), jnp.int32).at[:E].set(src.astype(jnp.int32))
            .reshape(NT, CHUNKS, CK))
    tgtg = (jnp.full((EP,), DUMMY, jnp.int32).at[:E].set(tgt.astype(jnp.int32))
            .reshape(NT, CHUNKS, CK))

    gsz = C // GROUPS
    G = jnp.kron(jnp.eye(GROUPS, dtype=_f32), jnp.ones((gsz, 1), _f32))
    P = (G @ G.T) / gsz
    gamma_row = gamma.reshape(1, C)
    beta_row = beta.reshape(1, C)
    b1row = b1.reshape(1, C)
    W2h = W2[1:]
    w20 = W2[0]
    dt = 1.0 / STEPS
    zeros_nc = jnp.zeros((NP, C), _f32)
    zrows = jnp.zeros((TROWS, H), _f32)

    def bias(tt):
        return (tt * w20 + b2).reshape(1, C)

    u0, u1 = _tce0(xp, W1, b1row)
    a0, a1 = _sc_seg(u0, u1, srcg, tgtg, zrows)
    u0, u1, h, ks = _tcf(1.0, 1.0, 0.0, a0, a1, zeros_nc, zeros_nc, Mp,
                         gamma_row, beta_row, P, W2h, bias(0.0))

    for i in range(STEPS):
        t = i * dt
        ks = zeros_nc
        for (wk, cz, cs), tb in [
            ((1.0, dt / 2, 0.0), t + dt / 2),
            ((2.0, dt / 2, 0.0), t + dt / 2),
            ((2.0, dt, 0.0), t + dt),
            ((1.0, 0.0, dt / 6), t + dt),
        ]:
            a0, a1 = _sc_seg(u0, u1, srcg, tgtg, zrows)
            if i == STEPS - 1 and cs != 0.0:
                return _tcsf(a0, a1, h, ks, Mp)[:N]
            hin = h
            u0, u1, z, ks = _tcf(wk, cz, cs, a0, a1, hin, ks, Mp,
                                 gamma_row, beta_row, P, W2h, bias(tb))
            if cs != 0.0:
                h = z
